# Initial kernel scaffold; baseline (speedup 1.0000x reference)
import jax
import jax.numpy as jnp
from jax.experimental import pallas as pl


def kernel(x):
    raise NotImplementedError("write your pallas kernel here")


# speedup vs baseline: 1.0085x; 1.0085x over previous
import functools

import jax
import jax.numpy as jnp
from jax.experimental import pallas as pl

_BLOCK_ROWS = 512


def _patch_tile(out_ref, rows, cols, row, col, value):
    tile = out_ref[rows, cols]
    r = jax.lax.broadcasted_iota(jnp.int32, tile.shape, 0)
    c = jax.lax.broadcasted_iota(jnp.int32, tile.shape, 1)
    mask = (r == row) & (c == col)
    out_ref[rows, cols] = jnp.where(mask, jnp.float32(value), tile)


def _copy_patch_body(in_ref, out_ref, *, n_cols, num_blocks, block_rows):
    out_ref[...] = in_ref[...]
    i = pl.program_id(0)

    @pl.when(i == 0)
    def _():
        _patch_tile(out_ref, pl.ds(0, 8), pl.ds(n_cols - 128, 128), 0, 126, 1.0)

    @pl.when(i == num_blocks - 1)
    def _():
        _patch_tile(out_ref, pl.ds(block_rows - 8, 8), pl.ds(0, 128), 7, 1, 5.0)


@jax.jit
def kernel(x):
    n_rows, n_cols = x.shape
    block_rows = _BLOCK_ROWS
    num_blocks = n_rows // block_rows
    body = functools.partial(
        _copy_patch_body,
        n_cols=n_cols,
        num_blocks=num_blocks,
        block_rows=block_rows,
    )
    return pl.pallas_call(
        body,
        grid=(num_blocks,),
        in_specs=[pl.BlockSpec((block_rows, n_cols), lambda i: (i, 0))],
        out_specs=pl.BlockSpec((block_rows, n_cols), lambda i: (i, 0)),
        out_shape=jax.ShapeDtypeStruct(x.shape, x.dtype),
    )(x)
